# Initial kernel scaffold; baseline (speedup 1.0000x reference)
import jax
import jax.numpy as jnp
from jax.experimental import pallas as pl


def kernel(x):
    raise NotImplementedError("write your pallas kernel here")


# speedup vs baseline: 1.0566x; 1.0566x over previous
import numpy as np
import jax
import jax.numpy as jnp
from jax.experimental import pallas as pl


_ROTS = ((13, 15, 26, 6), (17, 29, 16, 24))


def _np_threefry2x32(k0, k1, x0, x1):
    M = 0xFFFFFFFF
    ks = (k0, k1, k0 ^ k1 ^ 0x1BD11BDA)
    x0 = (x0 + ks[0]) & M
    x1 = (x1 + ks[1]) & M
    for r in range(5):
        for d in _ROTS[r % 2]:
            x0 = (x0 + x1) & M
            x1 = ((x1 << d) | (x1 >> (32 - d))) & M
            x1 ^= x0
        x0 = (x0 + ks[(r + 1) % 3]) & M
        x1 = (x1 + ks[(r + 2) % 3] + r + 1) & M
    return x0, x1


_SEED = (0, 42)
_K1 = _np_threefry2x32(_SEED[0], _SEED[1], 0, 0)
_K2 = _np_threefry2x32(_SEED[0], _SEED[1], 0, 1)

_MASK_T = int(np.ceil(np.float64(np.float32(0.07)) * (1 << 23)))
_REPL_T = 1 << 22

B, C, H, W = 64, 3, 224, 224
S = H * W
_R = 56
_GRID = (B, H // _R)


def _keystream(key, x1):
    k0, k1 = np.uint32(key[0]), np.uint32(key[1])
    ks2 = np.uint32(int(k0) ^ int(k1) ^ 0x1BD11BDA)
    ks = (k0, k1, ks2)
    x0 = jnp.full(x1.shape, k0, jnp.uint32)
    x1 = x1 + k1
    for r in range(5):
        for d in _ROTS[r % 2]:
            x0 = x0 + x1
            x1 = (x1 << np.uint32(d)) | (x1 >> np.uint32(32 - d))
            x1 = x1 ^ x0
        x0 = x0 + ks[(r + 1) % 3]
        x1 = x1 + np.uint32(int(ks[(r + 2) % 3]) + r + 1 & 0xFFFFFFFF)
    return x0 ^ x1


def _noise_kernel(x_ref, o_ref):
    b = pl.program_id(0)
    k = pl.program_id(1)
    row = jax.lax.broadcasted_iota(jnp.uint32, (_R, W), 0)
    col = jax.lax.broadcasted_iota(jnp.uint32, (_R, W), 1)
    s = (jnp.uint32(k * _R) + row) * np.uint32(W) + col

    bu = jnp.uint32(b)
    mask_bits = _keystream(_K1, bu * np.uint32(S) + s)
    mask = (mask_bits >> np.uint32(9)) < np.uint32(_MASK_T)

    for c in range(C):
        repl_bits = _keystream(_K2, (bu * np.uint32(C) + np.uint32(c)) * np.uint32(S) + s)
        one = (repl_bits >> np.uint32(9)) > np.uint32(_REPL_T)
        xc = x_ref[0, c, :, :]
        out = jnp.where(mask, jnp.where(one, jnp.float32(1.0), jnp.float32(0.0)),
                        jnp.clip(xc, 0.0, 1.0))
        o_ref[0, c, :, :] = out


def kernel(x):
    spec = pl.BlockSpec((1, C, _R, W), lambda b, k: (b, 0, k, 0))
    return pl.pallas_call(
        _noise_kernel,
        grid=_GRID,
        in_specs=[spec],
        out_specs=spec,
        out_shape=jax.ShapeDtypeStruct((B, C, H, W), jnp.float32),
    )(x)


# speedup vs baseline: 1.1431x; 1.0818x over previous
import numpy as np
import jax
import jax.numpy as jnp
from jax.experimental import pallas as pl


_ROTS = ((13, 15, 26, 6), (17, 29, 16, 24))


def _np_threefry2x32(k0, k1, x0, x1):
    M = 0xFFFFFFFF
    ks = (k0, k1, k0 ^ k1 ^ 0x1BD11BDA)
    x0 = (x0 + ks[0]) & M
    x1 = (x1 + ks[1]) & M
    for r in range(5):
        for d in _ROTS[r % 2]:
            x0 = (x0 + x1) & M
            x1 = ((x1 << d) | (x1 >> (32 - d))) & M
            x1 ^= x0
        x0 = (x0 + ks[(r + 1) % 3]) & M
        x1 = (x1 + ks[(r + 2) % 3] + r + 1) & M
    return x0, x1


_SEED = (0, 42)
_K1 = _np_threefry2x32(_SEED[0], _SEED[1], 0, 0)
_K2 = _np_threefry2x32(_SEED[0], _SEED[1], 0, 1)

_MASK_T = 587203 * 512
_REPL_T = (1 << 22 | 1) << 9

B, C, H, W = 64, 3, 224, 224
S = H * W
_R = 112
_GRID = (B, H // _R)


def _keystream(key, x1):
    k0, k1 = np.uint32(key[0]), np.uint32(key[1])
    ks2 = np.uint32(int(k0) ^ int(k1) ^ 0x1BD11BDA)
    ks = (k0, k1, ks2)
    x0 = jnp.full(x1.shape, k0, jnp.uint32)
    x1 = x1 + k1
    for r in range(5):
        for d in _ROTS[r % 2]:
            x0 = x0 + x1
            x1 = (x1 << np.uint32(d)) | (x1 >> np.uint32(32 - d))
            x1 = x1 ^ x0
        x0 = x0 + ks[(r + 1) % 3]
        x1 = x1 + np.uint32(int(ks[(r + 2) % 3]) + r + 1 & 0xFFFFFFFF)
    return x0 ^ x1


def _noise_kernel(x_ref, o_ref):
    b = pl.program_id(0)
    k = pl.program_id(1)
    row = jax.lax.broadcasted_iota(jnp.uint32, (_R, W), 0)
    col = jax.lax.broadcasted_iota(jnp.uint32, (_R, W), 1)
    s = (jnp.uint32(k * _R) + row) * np.uint32(W) + col

    bu = jnp.uint32(b)
    mask_bits = _keystream(_K1, bu * np.uint32(S) + s)
    mask = mask_bits < np.uint32(_MASK_T)

    for c in range(C):
        repl_bits = _keystream(_K2, (bu * np.uint32(C) + np.uint32(c)) * np.uint32(S) + s)
        one = repl_bits >= np.uint32(_REPL_T)
        xc = x_ref[0, c, :, :]
        out = jnp.where(mask, jnp.where(one, jnp.float32(1.0), jnp.float32(0.0)),
                        jnp.clip(xc, 0.0, 1.0))
        o_ref[0, c, :, :] = out


def kernel(x):
    spec = pl.BlockSpec((1, C, _R, W), lambda b, k: (b, 0, k, 0))
    return pl.pallas_call(
        _noise_kernel,
        grid=_GRID,
        in_specs=[spec],
        out_specs=spec,
        out_shape=jax.ShapeDtypeStruct((B, C, H, W), jnp.float32),
    )(x)


# speedup vs baseline: 1.1661x; 1.0202x over previous
import numpy as np
import jax
import jax.numpy as jnp
from jax.experimental import pallas as pl


_ROTS = ((13, 15, 26, 6), (17, 29, 16, 24))


def _np_threefry2x32(k0, k1, x0, x1):
    M = 0xFFFFFFFF
    ks = (k0, k1, k0 ^ k1 ^ 0x1BD11BDA)
    x0 = (x0 + ks[0]) & M
    x1 = (x1 + ks[1]) & M
    for r in range(5):
        for d in _ROTS[r % 2]:
            x0 = (x0 + x1) & M
            x1 = ((x1 << d) | (x1 >> (32 - d))) & M
            x1 ^= x0
        x0 = (x0 + ks[(r + 1) % 3]) & M
        x1 = (x1 + ks[(r + 2) % 3] + r + 1) & M
    return x0, x1


_SEED = (0, 42)
_K1 = _np_threefry2x32(_SEED[0], _SEED[1], 0, 0)
_K2 = _np_threefry2x32(_SEED[0], _SEED[1], 0, 1)

_MASK_T = 587203 * 512
_REPL_T = (1 << 22 | 1) << 9

B, C, H, W = 64, 3, 224, 224
S = H * W
_R = 224
_GRID = (B, H // _R)


def _keystream(key, x1):
    k0, k1 = np.uint32(key[0]), np.uint32(key[1])
    ks2 = np.uint32(int(k0) ^ int(k1) ^ 0x1BD11BDA)
    ks = (k0, k1, ks2)
    x0 = jnp.full(x1.shape, k0, jnp.uint32)
    x1 = x1 + k1
    for r in range(5):
        for d in _ROTS[r % 2]:
            x0 = x0 + x1
            x1 = (x1 << np.uint32(d)) | (x1 >> np.uint32(32 - d))
            x1 = x1 ^ x0
        x0 = x0 + ks[(r + 1) % 3]
        x1 = x1 + np.uint32(int(ks[(r + 2) % 3]) + r + 1 & 0xFFFFFFFF)
    return x0 ^ x1


def _noise_kernel(x_ref, o_ref):
    b = pl.program_id(0)
    k = pl.program_id(1)
    row = jax.lax.broadcasted_iota(jnp.uint32, (_R, W), 0)
    col = jax.lax.broadcasted_iota(jnp.uint32, (_R, W), 1)
    s = (jnp.uint32(k * _R) + row) * np.uint32(W) + col

    bu = jnp.uint32(b)
    mask_bits = _keystream(_K1, bu * np.uint32(S) + s)
    mask = mask_bits < np.uint32(_MASK_T)

    for c in range(C):
        repl_bits = _keystream(_K2, (bu * np.uint32(C) + np.uint32(c)) * np.uint32(S) + s)
        one = repl_bits >= np.uint32(_REPL_T)
        xc = x_ref[0, c, :, :]
        out = jnp.where(mask, jnp.where(one, jnp.float32(1.0), jnp.float32(0.0)),
                        jnp.clip(xc, 0.0, 1.0))
        o_ref[0, c, :, :] = out


def kernel(x):
    spec = pl.BlockSpec((1, C, _R, W), lambda b, k: (b, 0, k, 0))
    return pl.pallas_call(
        _noise_kernel,
        grid=_GRID,
        in_specs=[spec],
        out_specs=spec,
        out_shape=jax.ShapeDtypeStruct((B, C, H, W), jnp.float32),
    )(x)
